# baseline (speedup 1.0000x reference)
import functools

import jax
import jax.numpy as jnp
from jax import lax
from jax.experimental import pallas as pl
from jax.experimental.pallas import tpu as pltpu
from jax.experimental.pallas import tpu_sc as plsc

B = 4096
L = 200
D = 64
C = 1000
C_PAD = 1024

NC, NS = 2, 16
NW = NC * NS
PER_W = B // NW
CE = 4
CHUNK_R = CE * L
NCHUNK = PER_W // CE
NLANE = D // 16


def _sc_sums(x_flat, table):
    mesh = plsc.VectorSubcoreMesh(
        core_axis_name="c", subcore_axis_name="s", num_cores=NC, num_subcores=NS
    )

    @functools.partial(
        pl.kernel,
        out_type=jax.ShapeDtypeStruct((B, D), jnp.float32),
        mesh=mesh,
        scratch_types=[
            pltpu.VMEM((CHUNK_R,), jnp.int32),
            pltpu.VMEM((CHUNK_R, D), jnp.float32),
            pltpu.VMEM((PER_W, D), jnp.float32),
            pltpu.SemaphoreType.DMA,
        ],
        compiler_params=pltpu.CompilerParams(use_tc_tiling_on_sc=False),
    )
    def k(x_hbm, tab_hbm, out_hbm, idx_v, rows_v, acc_v, sem):
        wid = lax.axis_index("s") * NC + lax.axis_index("c")
        base_e = wid * PER_W

        def chunk_body(g, carry):
            pltpu.sync_copy(
                x_hbm.at[pl.ds(base_e * L + g * CHUNK_R, CHUNK_R)], idx_v
            )
            pltpu.async_copy(tab_hbm.at[idx_v], rows_v, sem).wait()

            def l_body(li, accs):
                out = []
                for e in range(CE):
                    for d4 in range(NLANE):
                        r0 = rows_v[e * L + 2 * li, pl.ds(16 * d4, 16)]
                        r1 = rows_v[e * L + 2 * li + 1, pl.ds(16 * d4, 16)]
                        out.append(accs[e * NLANE + d4] + (r0 + r1))
                return tuple(out)

            zeros = tuple(
                jnp.zeros((16,), jnp.float32) for _ in range(CE * NLANE)
            )
            accs = lax.fori_loop(0, L // 2, l_body, zeros)
            for e in range(CE):
                for d4 in range(NLANE):
                    acc_v[g * CE + e, pl.ds(16 * d4, 16)] = accs[e * NLANE + d4]
            return carry

        lax.fori_loop(0, NCHUNK, chunk_body, 0)
        pltpu.sync_copy(acc_v, out_hbm.at[pl.ds(base_e, PER_W)])

    return k(x_flat, table)


def _tc_head(sums, w_pad, b_pad):
    def body(x_ref, w_ref, b_ref, o_ref):
        x = x_ref[...] * jnp.float32(1.0 / L)
        o_ref[...] = (
            lax.dot_general(
                x, w_ref[...], (((1,), (1,)), ((), ())),
                preferred_element_type=jnp.float32,
            )
            + b_ref[...]
        )

    mblk = 512
    return pl.pallas_call(
        body,
        grid=(B // mblk,),
        in_specs=[
            pl.BlockSpec((mblk, D), lambda i: (i, 0)),
            pl.BlockSpec((C_PAD, D), lambda i: (0, 0)),
            pl.BlockSpec((1, C_PAD), lambda i: (0, 0)),
        ],
        out_specs=pl.BlockSpec((mblk, C_PAD), lambda i: (i, 0)),
        out_shape=jax.ShapeDtypeStruct((B, C_PAD), jnp.float32),
    )(sums, w_pad, b_pad)


def kernel(x_data, table, W, b):
    x_flat = x_data.reshape(-1).astype(jnp.int32)
    sums = _sc_sums(x_flat, table)
    w_pad = jnp.pad(W, ((0, C_PAD - C), (0, 0)))
    b_pad = jnp.pad(b, (0, C_PAD - C)).reshape(1, C_PAD)
    pred = _tc_head(sums, w_pad, b_pad)
    return pred[:, :C]


# speedup vs baseline: 1.0752x; 1.0752x over previous
import functools

import jax
import jax.numpy as jnp
import numpy as np
from jax import lax
from jax.experimental import pallas as pl
from jax.experimental.pallas import tpu as pltpu
from jax.experimental.pallas import tpu_sc as plsc

B = 4096
L = 200
D = 64
DW = D // 2
C = 1000
C_PAD = 1024

NC, NS = 2, 16
NW = NC * NS
PER_W = B // NW
CE = 4
CHUNK_R = CE * L
NCHUNK = PER_W // CE

_PERM = np.concatenate(
    [np.arange(0, 16), np.arange(32, 48), np.arange(16, 32), np.arange(48, 64)]
)


def _sc_sums(x_flat, tab32):
    mesh = plsc.VectorSubcoreMesh(
        core_axis_name="c", subcore_axis_name="s", num_cores=NC, num_subcores=NS
    )

    @functools.partial(
        pl.kernel,
        out_type=jax.ShapeDtypeStruct((B, D), jnp.float32),
        mesh=mesh,
        scratch_types=[
            pltpu.VMEM((CHUNK_R,), jnp.int32),
            pltpu.VMEM((CHUNK_R,), jnp.int32),
            pltpu.VMEM((CHUNK_R, DW), jnp.int32),
            pltpu.VMEM((CHUNK_R, DW), jnp.int32),
            pltpu.VMEM((PER_W, D), jnp.float32),
            pltpu.SemaphoreType.DMA,
            pltpu.SemaphoreType.DMA,
        ],
        compiler_params=pltpu.CompilerParams(use_tc_tiling_on_sc=False),
    )
    def k(x_hbm, tab_hbm, out_hbm, idx0, idx1, rows0, rows1, acc_v, sem0, sem1):
        wid = lax.axis_index("s") * NC + lax.axis_index("c")
        base_e = wid * PER_W
        idx_b = (idx0, idx1)
        rows_b = (rows0, rows1)
        sem_b = (sem0, sem1)

        def start(g, b):
            pltpu.sync_copy(
                x_hbm.at[pl.ds(base_e * L + g * CHUNK_R, CHUNK_R)], idx_b[b]
            )
            pltpu.async_copy(tab_hbm.at[idx_b[b]], rows_b[b], sem_b[b])

        def accum(g, b):
            rows = rows_b[b]
            mask = jnp.full((16,), -65536, jnp.int32)

            def l_body(li, accs):
                out = list(accs)
                for e in range(CE):
                    for q in range(2):
                        v = rows[e * L + li, pl.ds(16 * q, 16)]
                        lo = lax.bitcast_convert_type(v * 65536, jnp.float32)
                        hi = lax.bitcast_convert_type(v & mask, jnp.float32)
                        out[4 * e + 2 * q] = out[4 * e + 2 * q] + lo
                        out[4 * e + 2 * q + 1] = out[4 * e + 2 * q + 1] + hi
                return tuple(out)

            zeros = tuple(jnp.zeros((16,), jnp.float32) for _ in range(4 * CE))
            accs = lax.fori_loop(0, L, l_body, zeros)
            for e in range(CE):
                for p in range(4):
                    acc_v[g * CE + e, pl.ds(16 * p, 16)] = accs[4 * e + p]

        start(0, 0)
        start(1, 1)

        def pair_body(gp, carry):
            for b in range(2):
                g = 2 * gp + b
                pltpu.make_async_copy(
                    tab_hbm.at[idx_b[b]], rows_b[b], sem_b[b]
                ).wait()
                accum(g, b)

                @pl.when(g + 2 < NCHUNK)
                def _():
                    start(g + 2, b)

            return carry

        lax.fori_loop(0, NCHUNK // 2, pair_body, 0)
        pltpu.sync_copy(acc_v, out_hbm.at[pl.ds(base_e, PER_W)])

    return k(x_flat, tab32)


def _tc_head(sums, w_perm_pad, b_pad):
    def body(w_ref, x_ref, b_ref, o_ref):
        x = x_ref[...] * jnp.float32(1.0 / L)
        o_ref[...] = (
            lax.dot_general(
                w_ref[...], x, (((1,), (1,)), ((), ())),
                preferred_element_type=jnp.float32,
            )
            + b_ref[...]
        )

    mblk = 512
    return pl.pallas_call(
        body,
        grid=(B // mblk,),
        in_specs=[
            pl.BlockSpec((C_PAD, D), lambda i: (0, 0)),
            pl.BlockSpec((mblk, D), lambda i: (i, 0)),
            pl.BlockSpec((C_PAD, 1), lambda i: (0, 0)),
        ],
        out_specs=pl.BlockSpec((C_PAD, mblk), lambda i: (0, i)),
        out_shape=jax.ShapeDtypeStruct((C_PAD, B), jnp.float32),
    )(w_perm_pad, sums, b_pad)


V = 1000000
VBLK = 8192


def _tc_pack(table_t):
    def body(x_ref, o_ref):
        xt = lax.transpose(x_ref[...], (1, 0))
        u = lax.bitcast_convert_type(xt, jnp.uint32)
        rne = lambda v: (v + 0x7FFF + ((v >> 16) & 1)) >> 16
        o_ref[...] = lax.bitcast_convert_type(
            rne(u[:, :DW]) | (rne(u[:, DW:]) << 16), jnp.int32
        )

    grid = (V + VBLK - 1) // VBLK
    return pl.pallas_call(
        body,
        grid=(grid,),
        in_specs=[pl.BlockSpec((D, VBLK), lambda i: (0, i))],
        out_specs=pl.BlockSpec((VBLK, DW), lambda i: (i, 0)),
        out_shape=jax.ShapeDtypeStruct((V, DW), jnp.int32),
    )(table_t)


def kernel(x_data, table, W, b):
    x_flat = x_data.reshape(-1).astype(jnp.int32)
    tab32 = _tc_pack(table.T)
    sums = _sc_sums(x_flat, tab32)
    w_perm_pad = jnp.pad(W[:, _PERM], ((0, C_PAD - C), (0, 0)))
    b_pad = jnp.pad(b, (0, C_PAD - C)).reshape(C_PAD, 1)
    pred_t = _tc_head(sums, w_perm_pad, b_pad)
    return pred_t[:C].T


# speedup vs baseline: 1.8536x; 1.7239x over previous
import functools

import jax
import jax.numpy as jnp
import numpy as np
from jax import lax
from jax.experimental import pallas as pl
from jax.experimental.pallas import tpu as pltpu
from jax.experimental.pallas import tpu_sc as plsc

B = 4096
L = 200
D = 64
DW = D // 2
C = 1000
C_PAD = 1024

V = 1000000
VBLK = 8192
VSUB = VBLK // 4
NBLK = (V + VBLK - 1) // VBLK
V_PAD = NBLK * VBLK

NC, NS = 2, 16
NW = NC * NS
PER_W = B // NW
CE = 4
CHUNK_R = CE * L
NCHUNK = PER_W // CE

_PERM = np.concatenate(
    [np.arange(0, 16), np.arange(32, 48), np.arange(16, 32), np.arange(48, 64)]
)


def _tc_pack(table_t):
    def body(x_ref, o_ref):
        outs = []
        for c in range(4):
            xt = lax.transpose(x_ref[:, VSUB * c:VSUB * (c + 1)], (1, 0))
            u = lax.bitcast_convert_type(xt, jnp.uint32)
            lo = (u[:, :DW] + 0x8000) >> 16
            hi = (u[:, DW:] + 0x8000) & jnp.uint32(0xFFFF0000)
            outs.append(lo | hi)
        o_ref[...] = lax.bitcast_convert_type(
            jnp.concatenate(outs, axis=1), jnp.int32
        )

    return pl.pallas_call(
        body,
        grid=(NBLK,),
        in_specs=[pl.BlockSpec((D, VBLK), lambda i: (0, i))],
        out_specs=pl.BlockSpec((VSUB, 128), lambda i: (i, 0)),
        out_shape=jax.ShapeDtypeStruct((NBLK * VSUB, 128), jnp.int32),
    )(table_t)


def _sc_sums(x_flat, tab32):
    mesh = plsc.VectorSubcoreMesh(
        core_axis_name="c", subcore_axis_name="s", num_cores=NC, num_subcores=NS
    )

    @functools.partial(
        pl.kernel,
        out_type=jax.ShapeDtypeStruct((B, D), jnp.float32),
        mesh=mesh,
        scratch_types=[
            pltpu.VMEM((CHUNK_R,), jnp.int32),
            pltpu.VMEM((CHUNK_R,), jnp.int32),
            pltpu.VMEM((CHUNK_R,), jnp.int32),
            pltpu.VMEM((CHUNK_R,), jnp.int32),
            pltpu.VMEM((CHUNK_R, DW), jnp.int32),
            pltpu.VMEM((CHUNK_R, DW), jnp.int32),
            pltpu.VMEM((PER_W, D), jnp.float32),
            pltpu.SemaphoreType.DMA,
            pltpu.SemaphoreType.DMA,
        ],
        compiler_params=pltpu.CompilerParams(use_tc_tiling_on_sc=False),
    )
    def k(x_hbm, tab_hbm, out_hbm, raw0, raw1, idx0, idx1, rows0, rows1,
          acc_v, sem0, sem1):
        wid = lax.axis_index("s") * NC + lax.axis_index("c")
        base_e = wid * PER_W
        raw_b = (raw0, raw1)
        idx_b = (idx0, idx1)
        rows_b = (rows0, rows1)
        sem_b = (sem0, sem1)

        def start(g, b):
            raw, idx = raw_b[b], idx_b[b]
            pltpu.sync_copy(
                x_hbm.at[pl.ds(base_e * L + g * CHUNK_R, CHUNK_R)], raw
            )

            def tr_body(j, carry):
                r = raw[pl.ds(16 * j, 16)]
                idx[pl.ds(16 * j, 16)] = (
                    (r & ~(VBLK - 1))
                    | ((r & (VSUB - 1)) << 2)
                    | ((r >> 11) & 3)
                )
                return carry

            lax.fori_loop(0, CHUNK_R // 16, tr_body, 0)
            pltpu.async_copy(tab_hbm.at[idx], rows_b[b], sem_b[b])

        def accum(g, b):
            rows = rows_b[b]
            mask = jnp.full((16,), -65536, jnp.int32)

            def l_body(li, accs):
                out = list(accs)
                for e in range(CE):
                    for q in range(2):
                        v = rows[e * L + li, pl.ds(16 * q, 16)]
                        lo = lax.bitcast_convert_type(v * 65536, jnp.float32)
                        hi = lax.bitcast_convert_type(v & mask, jnp.float32)
                        out[4 * e + 2 * q] = out[4 * e + 2 * q] + lo
                        out[4 * e + 2 * q + 1] = out[4 * e + 2 * q + 1] + hi
                return tuple(out)

            zeros = tuple(jnp.zeros((16,), jnp.float32) for _ in range(4 * CE))
            accs = lax.fori_loop(0, L, l_body, zeros)
            for e in range(CE):
                for p in range(4):
                    acc_v[g * CE + e, pl.ds(16 * p, 16)] = accs[4 * e + p]

        start(0, 0)
        start(1, 1)

        def pair_body(gp, carry):
            for b in range(2):
                g = 2 * gp + b
                pltpu.make_async_copy(
                    tab_hbm.at[idx_b[b]], rows_b[b], sem_b[b]
                ).wait()
                accum(g, b)

                @pl.when(g + 2 < NCHUNK)
                def _():
                    start(g + 2, b)

            return carry

        lax.fori_loop(0, NCHUNK // 2, pair_body, 0)
        pltpu.sync_copy(acc_v, out_hbm.at[pl.ds(base_e, PER_W)])

    return k(x_flat, tab32)


def _tc_head(sums, w_perm_pad, b_pad):
    def body(w_ref, x_ref, b_ref, o_ref):
        x = x_ref[...] * jnp.float32(1.0 / L)
        o_ref[...] = (
            lax.dot_general(
                w_ref[...], x, (((1,), (1,)), ((), ())),
                preferred_element_type=jnp.float32,
            )
            + b_ref[...]
        )

    mblk = 512
    return pl.pallas_call(
        body,
        grid=(B // mblk,),
        in_specs=[
            pl.BlockSpec((C_PAD, D), lambda i: (0, 0)),
            pl.BlockSpec((mblk, D), lambda i: (i, 0)),
            pl.BlockSpec((C_PAD, 1), lambda i: (0, 0)),
        ],
        out_specs=pl.BlockSpec((C_PAD, mblk), lambda i: (0, i)),
        out_shape=jax.ShapeDtypeStruct((C_PAD, B), jnp.float32),
    )(w_perm_pad, sums, b_pad)


def kernel(x_data, table, W, b):
    x_flat = x_data.reshape(-1).astype(jnp.int32)
    packed = _tc_pack(table.T)
    tab32 = packed.reshape(NBLK * VBLK, DW)
    sums = _sc_sums(x_flat, tab32)
    w_perm_pad = jnp.pad(W[:, _PERM], ((0, C_PAD - C), (0, 0)))
    b_pad = jnp.pad(b, (0, C_PAD - C)).reshape(C_PAD, 1)
    pred_t = _tc_head(sums, w_perm_pad, b_pad)
    return pred_t[:C].T


# speedup vs baseline: 2.9219x; 1.5763x over previous
import functools

import jax
import jax.numpy as jnp
import numpy as np
from jax import lax
from jax.experimental import pallas as pl
from jax.experimental.pallas import tpu as pltpu
from jax.experimental.pallas import tpu_sc as plsc

B = 4096
L = 200
D = 64
DW = D // 2
C = 1000
C_PAD = 1024

V = 1000000
VBLK = 8192
VSUB = VBLK // 4
NBLK = (V + VBLK - 1) // VBLK
V_PAD = NBLK * VBLK

NC, NS = 2, 16
NW = NC * NS
PER_W = B // NW
CE = 4
CHUNK_R = CE * L
NCHUNK = PER_W // CE

_PERM = np.concatenate(
    [np.arange(0, 16), np.arange(32, 48), np.arange(16, 32), np.arange(48, 64)]
)


def _tc_pack(table_t):
    def body(x_ref, o_ref):
        u = lax.bitcast_convert_type(x_ref[...], jnp.uint32)
        lo = (u[:DW, :] + 0x8000) >> 16
        hi = (u[DW:, :] + 0x8000) & jnp.uint32(0xFFFF0000)
        p = lo | hi
        y = jnp.concatenate(
            [p[:, VSUB * c:VSUB * (c + 1)] for c in range(4)], axis=0
        )
        o_ref[...] = lax.bitcast_convert_type(
            lax.transpose(y, (1, 0)), jnp.int32
        )

    return pl.pallas_call(
        body,
        grid=(NBLK,),
        in_specs=[pl.BlockSpec((D, VBLK), lambda i: (0, i))],
        out_specs=pl.BlockSpec((VSUB, 128), lambda i: (i, 0)),
        out_shape=jax.ShapeDtypeStruct((NBLK * VSUB, 128), jnp.int32),
    )(table_t)


def _sc_sums(x_flat, tab32):
    mesh = plsc.VectorSubcoreMesh(
        core_axis_name="c", subcore_axis_name="s", num_cores=NC, num_subcores=NS
    )

    @functools.partial(
        pl.kernel,
        out_type=jax.ShapeDtypeStruct((B, D), jnp.float32),
        mesh=mesh,
        scratch_types=[
            pltpu.VMEM((CHUNK_R,), jnp.int32),
            pltpu.VMEM((CHUNK_R,), jnp.int32),
            pltpu.VMEM((CHUNK_R,), jnp.int32),
            pltpu.VMEM((CHUNK_R,), jnp.int32),
            pltpu.VMEM((CHUNK_R, DW), jnp.int32),
            pltpu.VMEM((CHUNK_R, DW), jnp.int32),
            pltpu.VMEM((PER_W, D), jnp.float32),
            pltpu.SemaphoreType.DMA,
            pltpu.SemaphoreType.DMA,
        ],
        compiler_params=pltpu.CompilerParams(use_tc_tiling_on_sc=False),
    )
    def k(x_hbm, tab_hbm, out_hbm, raw0, raw1, idx0, idx1, rows0, rows1,
          acc_v, sem0, sem1):
        wid = lax.axis_index("s") * NC + lax.axis_index("c")
        base_e = wid * PER_W
        raw_b = (raw0, raw1)
        idx_b = (idx0, idx1)
        rows_b = (rows0, rows1)
        sem_b = (sem0, sem1)

        def start(g, b):
            raw, idx = raw_b[b], idx_b[b]
            pltpu.sync_copy(
                x_hbm.at[pl.ds(base_e * L + g * CHUNK_R, CHUNK_R)], raw
            )

            def tr_body(j, carry):
                r = raw[pl.ds(16 * j, 16)]
                idx[pl.ds(16 * j, 16)] = (
                    (r & ~(VBLK - 1))
                    | ((r & (VSUB - 1)) << 2)
                    | ((r >> 11) & 3)
                )
                return carry

            lax.fori_loop(0, CHUNK_R // 16, tr_body, 0)
            pltpu.async_copy(tab_hbm.at[idx], rows_b[b], sem_b[b])

        def accum(g, b):
            rows = rows_b[b]
            mask = jnp.full((16,), -65536, jnp.int32)

            def l_body(li, accs):
                out = list(accs)
                for e in range(CE):
                    for q in range(2):
                        v = rows[e * L + li, pl.ds(16 * q, 16)]
                        lo = lax.bitcast_convert_type(v * 65536, jnp.float32)
                        hi = lax.bitcast_convert_type(v & mask, jnp.float32)
                        out[4 * e + 2 * q] = out[4 * e + 2 * q] + lo
                        out[4 * e + 2 * q + 1] = out[4 * e + 2 * q + 1] + hi
                return tuple(out)

            zeros = tuple(jnp.zeros((16,), jnp.float32) for _ in range(4 * CE))
            accs = lax.fori_loop(0, L, l_body, zeros)
            for e in range(CE):
                for p in range(4):
                    acc_v[g * CE + e, pl.ds(16 * p, 16)] = accs[4 * e + p]

        start(0, 0)
        start(1, 1)

        def pair_body(gp, carry):
            for b in range(2):
                g = 2 * gp + b
                pltpu.make_async_copy(
                    tab_hbm.at[idx_b[b]], rows_b[b], sem_b[b]
                ).wait()
                accum(g, b)

                @pl.when(g + 2 < NCHUNK)
                def _():
                    start(g + 2, b)

            return carry

        lax.fori_loop(0, NCHUNK // 2, pair_body, 0)
        pltpu.sync_copy(acc_v, out_hbm.at[pl.ds(base_e, PER_W)])

    return k(x_flat, tab32)


def _tc_head(sums, w_perm_pad, b_pad):
    def body(w_ref, x_ref, b_ref, o_ref):
        x = x_ref[...] * jnp.float32(1.0 / L)
        o_ref[...] = (
            lax.dot_general(
                w_ref[...], x, (((1,), (1,)), ((), ())),
                preferred_element_type=jnp.float32,
            )
            + b_ref[...]
        )

    mblk = 512
    return pl.pallas_call(
        body,
        grid=(B // mblk,),
        in_specs=[
            pl.BlockSpec((C_PAD, D), lambda i: (0, 0)),
            pl.BlockSpec((mblk, D), lambda i: (i, 0)),
            pl.BlockSpec((C_PAD, 1), lambda i: (0, 0)),
        ],
        out_specs=pl.BlockSpec((C_PAD, mblk), lambda i: (0, i)),
        out_shape=jax.ShapeDtypeStruct((C_PAD, B), jnp.float32),
    )(w_perm_pad, sums, b_pad)


def kernel(x_data, table, W, b):
    x_flat = x_data.reshape(-1).astype(jnp.int32)
    packed = _tc_pack(table.T)
    tab32 = packed.reshape(NBLK * VBLK, DW)
    sums = _sc_sums(x_flat, tab32)
    w_perm_pad = jnp.pad(W[:, _PERM], ((0, C_PAD - C), (0, 0)))
    b_pad = jnp.pad(b, (0, C_PAD - C)).reshape(C_PAD, 1)
    pred_t = _tc_head(sums, w_perm_pad, b_pad)
    return pred_t[:C].T


# speedup vs baseline: 3.2954x; 1.1278x over previous
import functools

import jax
import jax.numpy as jnp
import numpy as np
from jax import lax
from jax.experimental import pallas as pl
from jax.experimental.pallas import tpu as pltpu
from jax.experimental.pallas import tpu_sc as plsc

B = 4096
L = 200
D = 64
DW = D // 2
C = 1000
C_PAD = 1024

V = 1000000
VBLK = 16384
VSUB = VBLK // 4
NBLK = (V + VBLK - 1) // VBLK
V_PAD = NBLK * VBLK

NC, NS = 2, 16
NW = NC * NS
PER_W = B // NW
CE = 4
CHUNK_R = CE * L
NCHUNK = PER_W // CE

_PERM = np.concatenate(
    [np.arange(0, 16), np.arange(32, 48), np.arange(16, 32), np.arange(48, 64)]
)


def _tc_pack(table_t):
    def body(x_ref, o_ref):
        u = lax.bitcast_convert_type(x_ref[...], jnp.uint32)
        lo = (u[:DW, :] + 0x8000) >> 16
        hi = (u[DW:, :] + 0x8000) & jnp.uint32(0xFFFF0000)
        p = lo | hi
        y = jnp.concatenate(
            [p[:, VSUB * c:VSUB * (c + 1)] for c in range(4)], axis=0
        )
        o_ref[...] = lax.bitcast_convert_type(
            lax.transpose(y, (1, 0)), jnp.int32
        )

    return pl.pallas_call(
        body,
        grid=(NBLK,),
        in_specs=[pl.BlockSpec((D, VBLK), lambda i: (0, i))],
        out_specs=pl.BlockSpec((VSUB, 128), lambda i: (i, 0)),
        out_shape=jax.ShapeDtypeStruct((NBLK * VSUB, 128), jnp.int32),
    )(table_t)


def _sc_sums(x_flat, tab32):
    mesh = plsc.VectorSubcoreMesh(
        core_axis_name="c", subcore_axis_name="s", num_cores=NC, num_subcores=NS
    )

    @functools.partial(
        pl.kernel,
        out_type=jax.ShapeDtypeStruct((B, D), jnp.float32),
        mesh=mesh,
        scratch_types=[
            pltpu.VMEM((CHUNK_R,), jnp.int32),
            pltpu.VMEM((CHUNK_R,), jnp.int32),
            pltpu.VMEM((CHUNK_R,), jnp.int32),
            pltpu.VMEM((CHUNK_R,), jnp.int32),
            pltpu.VMEM((CHUNK_R, DW), jnp.int32),
            pltpu.VMEM((CHUNK_R, DW), jnp.int32),
            pltpu.VMEM((PER_W, D), jnp.float32),
            pltpu.SemaphoreType.DMA,
            pltpu.SemaphoreType.DMA,
        ],
        compiler_params=pltpu.CompilerParams(use_tc_tiling_on_sc=False),
    )
    def k(x_hbm, tab_hbm, out_hbm, raw0, raw1, idx0, idx1, rows0, rows1,
          acc_v, sem0, sem1):
        wid = lax.axis_index("s") * NC + lax.axis_index("c")
        base_e = wid * PER_W
        raw_b = (raw0, raw1)
        idx_b = (idx0, idx1)
        rows_b = (rows0, rows1)
        sem_b = (sem0, sem1)

        def start(g, b):
            raw, idx = raw_b[b], idx_b[b]
            pltpu.sync_copy(
                x_hbm.at[pl.ds(base_e * L + g * CHUNK_R, CHUNK_R)], raw
            )

            def tr_body(j, carry):
                r = raw[pl.ds(16 * j, 16)]
                idx[pl.ds(16 * j, 16)] = (
                    (r & ~(VBLK - 1))
                    | ((r & (VSUB - 1)) << 2)
                    | ((r >> 12) & 3)
                )
                return carry

            lax.fori_loop(0, CHUNK_R // 16, tr_body, 0)
            pltpu.async_copy(tab_hbm.at[idx], rows_b[b], sem_b[b])

        def accum(g, b):
            rows = rows_b[b]
            mask = jnp.full((16,), -65536, jnp.int32)

            def l_body(li, accs):
                out = list(accs)
                for e in range(CE):
                    for q in range(2):
                        v = rows[e * L + li, pl.ds(16 * q, 16)]
                        lo = lax.bitcast_convert_type(v * 65536, jnp.float32)
                        hi = lax.bitcast_convert_type(v & mask, jnp.float32)
                        out[4 * e + 2 * q] = out[4 * e + 2 * q] + lo
                        out[4 * e + 2 * q + 1] = out[4 * e + 2 * q + 1] + hi
                return tuple(out)

            zeros = tuple(jnp.zeros((16,), jnp.float32) for _ in range(4 * CE))
            accs = lax.fori_loop(0, L, l_body, zeros)
            for e in range(CE):
                for p in range(4):
                    acc_v[g * CE + e, pl.ds(16 * p, 16)] = accs[4 * e + p]

        start(0, 0)
        start(1, 1)

        def pair_body(gp, carry):
            for b in range(2):
                g = 2 * gp + b
                pltpu.make_async_copy(
                    tab_hbm.at[idx_b[b]], rows_b[b], sem_b[b]
                ).wait()
                accum(g, b)

                @pl.when(g + 2 < NCHUNK)
                def _():
                    start(g + 2, b)

            return carry

        lax.fori_loop(0, NCHUNK // 2, pair_body, 0)
        pltpu.sync_copy(acc_v, out_hbm.at[pl.ds(base_e, PER_W)])

    return k(x_flat, tab32)


def _tc_head(sums, w_perm_pad, b_pad):
    def body(w_ref, x_ref, b_ref, o_ref):
        x = x_ref[...] * jnp.float32(1.0 / L)
        o_ref[...] = (
            lax.dot_general(
                w_ref[...], x, (((1,), (1,)), ((), ())),
                preferred_element_type=jnp.float32,
            )
            + b_ref[...]
        )

    mblk = 512
    return pl.pallas_call(
        body,
        grid=(B // mblk,),
        in_specs=[
            pl.BlockSpec((C_PAD, D), lambda i: (0, 0)),
            pl.BlockSpec((mblk, D), lambda i: (i, 0)),
            pl.BlockSpec((C_PAD, 1), lambda i: (0, 0)),
        ],
        out_specs=pl.BlockSpec((C_PAD, mblk), lambda i: (0, i)),
        out_shape=jax.ShapeDtypeStruct((C_PAD, B), jnp.float32),
    )(w_perm_pad, sums, b_pad)


def kernel(x_data, table, W, b):
    x_flat = x_data.reshape(-1).astype(jnp.int32)
    packed = _tc_pack(table.T)
    tab32 = packed.reshape(NBLK * VBLK, DW)
    sums = _sc_sums(x_flat, tab32)
    w_perm_pad = jnp.pad(W[:, _PERM], ((0, C_PAD - C), (0, 0)))
    b_pad = jnp.pad(b, (0, C_PAD - C)).reshape(C_PAD, 1)
    pred_t = _tc_head(sums, w_perm_pad, b_pad)
    return pred_t[:C].T


# speedup vs baseline: 3.3766x; 1.0246x over previous
import functools

import jax
import jax.numpy as jnp
import numpy as np
from jax import lax
from jax.experimental import pallas as pl
from jax.experimental.pallas import tpu as pltpu
from jax.experimental.pallas import tpu_sc as plsc

B = 4096
L = 200
D = 64
DW = D // 2
C = 1000
C_PAD = 1024

V = 1000000
VBLK = 32768
VSUB = VBLK // 4
NBLK = (V + VBLK - 1) // VBLK
V_PAD = NBLK * VBLK

NC, NS = 2, 16
NW = NC * NS
PER_W = B // NW
CE = 4
CHUNK_R = CE * L
NCHUNK = PER_W // CE

_PERM = np.concatenate(
    [np.arange(0, 16), np.arange(32, 48), np.arange(16, 32), np.arange(48, 64)]
)


def _tc_pack(table_t):
    def body(x_ref, o_ref):
        u = lax.bitcast_convert_type(x_ref[...], jnp.uint32)
        lo = (u[:DW, :] + 0x8000) >> 16
        hi = (u[DW:, :] + 0x8000) & jnp.uint32(0xFFFF0000)
        p = lo | hi
        y = jnp.concatenate(
            [p[:, VSUB * c:VSUB * (c + 1)] for c in range(4)], axis=0
        )
        o_ref[...] = lax.bitcast_convert_type(
            lax.transpose(y, (1, 0)), jnp.int32
        )

    return pl.pallas_call(
        body,
        grid=(NBLK,),
        in_specs=[pl.BlockSpec((D, VBLK), lambda i: (0, i))],
        out_specs=pl.BlockSpec((VSUB, 128), lambda i: (i, 0)),
        out_shape=jax.ShapeDtypeStruct((NBLK * VSUB, 128), jnp.int32),
    )(table_t)


def _sc_sums(x_flat, tab32):
    mesh = plsc.VectorSubcoreMesh(
        core_axis_name="c", subcore_axis_name="s", num_cores=NC, num_subcores=NS
    )

    @functools.partial(
        pl.kernel,
        out_type=jax.ShapeDtypeStruct((B, D), jnp.float32),
        mesh=mesh,
        scratch_types=[
            pltpu.VMEM((CHUNK_R,), jnp.int32),
            pltpu.VMEM((CHUNK_R,), jnp.int32),
            pltpu.VMEM((CHUNK_R,), jnp.int32),
            pltpu.VMEM((CHUNK_R,), jnp.int32),
            pltpu.VMEM((CHUNK_R, DW), jnp.int32),
            pltpu.VMEM((CHUNK_R, DW), jnp.int32),
            pltpu.VMEM((PER_W, D), jnp.float32),
            pltpu.SemaphoreType.DMA,
            pltpu.SemaphoreType.DMA,
        ],
        compiler_params=pltpu.CompilerParams(use_tc_tiling_on_sc=False),
    )
    def k(x_hbm, tab_hbm, out_hbm, raw0, raw1, idx0, idx1, rows0, rows1,
          acc_v, sem0, sem1):
        wid = lax.axis_index("s") * NC + lax.axis_index("c")
        base_e = wid * PER_W
        raw_b = (raw0, raw1)
        idx_b = (idx0, idx1)
        rows_b = (rows0, rows1)
        sem_b = (sem0, sem1)

        def start(g, b):
            raw, idx = raw_b[b], idx_b[b]
            pltpu.sync_copy(
                x_hbm.at[pl.ds(base_e * L + g * CHUNK_R, CHUNK_R)], raw
            )

            def tr_body(j, carry):
                r = raw[pl.ds(16 * j, 16)]
                idx[pl.ds(16 * j, 16)] = (
                    (r & ~(VBLK - 1))
                    | ((r & (VSUB - 1)) << 2)
                    | ((r >> 13) & 3)
                )
                return carry

            lax.fori_loop(0, CHUNK_R // 16, tr_body, 0)
            pltpu.async_copy(tab_hbm.at[idx], rows_b[b], sem_b[b])

        def accum(g, b):
            rows = rows_b[b]
            mask = jnp.full((16,), -65536, jnp.int32)

            def l_body(li, accs):
                out = list(accs)
                for e in range(CE):
                    for q in range(2):
                        v = rows[e * L + li, pl.ds(16 * q, 16)]
                        lo = lax.bitcast_convert_type(v * 65536, jnp.float32)
                        hi = lax.bitcast_convert_type(v & mask, jnp.float32)
                        out[4 * e + 2 * q] = out[4 * e + 2 * q] + lo
                        out[4 * e + 2 * q + 1] = out[4 * e + 2 * q + 1] + hi
                return tuple(out)

            zeros = tuple(jnp.zeros((16,), jnp.float32) for _ in range(4 * CE))
            accs = lax.fori_loop(0, L, l_body, zeros)
            for e in range(CE):
                for p in range(4):
                    acc_v[g * CE + e, pl.ds(16 * p, 16)] = accs[4 * e + p]

        start(0, 0)
        start(1, 1)

        def pair_body(gp, carry):
            for b in range(2):
                g = 2 * gp + b
                pltpu.make_async_copy(
                    tab_hbm.at[idx_b[b]], rows_b[b], sem_b[b]
                ).wait()
                accum(g, b)

                @pl.when(g + 2 < NCHUNK)
                def _():
                    start(g + 2, b)

            return carry

        lax.fori_loop(0, NCHUNK // 2, pair_body, 0)
        pltpu.sync_copy(acc_v, out_hbm.at[pl.ds(base_e, PER_W)])

    return k(x_flat, tab32)


def _tc_head(sums, w_perm_pad, b_pad):
    def body(w_ref, x_ref, b_ref, o_ref):
        x = x_ref[...] * jnp.float32(1.0 / L)
        o_ref[...] = (
            lax.dot_general(
                w_ref[...], x, (((1,), (1,)), ((), ())),
                preferred_element_type=jnp.float32,
            )
            + b_ref[...]
        )

    mblk = 512
    return pl.pallas_call(
        body,
        grid=(B // mblk,),
        in_specs=[
            pl.BlockSpec((C_PAD, D), lambda i: (0, 0)),
            pl.BlockSpec((mblk, D), lambda i: (i, 0)),
            pl.BlockSpec((C_PAD, 1), lambda i: (0, 0)),
        ],
        out_specs=pl.BlockSpec((C_PAD, mblk), lambda i: (0, i)),
        out_shape=jax.ShapeDtypeStruct((C_PAD, B), jnp.float32),
    )(w_perm_pad, sums, b_pad)


def kernel(x_data, table, W, b):
    x_flat = x_data.reshape(-1).astype(jnp.int32)
    packed = _tc_pack(table.T)
    tab32 = packed.reshape(NBLK * VBLK, DW)
    sums = _sc_sums(x_flat, tab32)
    w_perm_pad = jnp.pad(W[:, _PERM], ((0, C_PAD - C), (0, 0)))
    b_pad = jnp.pad(b, (0, C_PAD - C)).reshape(C_PAD, 1)
    pred_t = _tc_head(sums, w_perm_pad, b_pad)
    return pred_t[:C].T


# speedup vs baseline: 3.5613x; 1.0547x over previous
import functools

import jax
import jax.numpy as jnp
import numpy as np
from jax import lax
from jax.experimental import pallas as pl
from jax.experimental.pallas import tpu as pltpu
from jax.experimental.pallas import tpu_sc as plsc

B = 4096
L = 200
D = 64
DW = D // 2
C = 1000
C_PAD = 1024

V = 1000000
VBLK = 32768
VSUB = VBLK // 4
NBLK = (V + VBLK - 1) // VBLK
V_PAD = NBLK * VBLK

NC, NS = 2, 16
NW = NC * NS
PER_W = B // NW
CE = 4
CHUNK_R = CE * L
NCHUNK = PER_W // CE

_PERM = np.concatenate(
    [np.arange(0, 16), np.arange(32, 48), np.arange(16, 32), np.arange(48, 64)]
)


def _tc_pack(table_t):
    def body(x_ref, o_ref):
        u = lax.bitcast_convert_type(x_ref[...], jnp.uint32)
        lo = (u[:DW, :] + 0x8000) >> 16
        hi = (u[DW:, :] + 0x8000) & jnp.uint32(0xFFFF0000)
        p = lo | hi
        y = jnp.concatenate(
            [p[:, VSUB * c:VSUB * (c + 1)] for c in range(4)], axis=0
        )
        o_ref[...] = lax.bitcast_convert_type(
            lax.transpose(y, (1, 0)), jnp.int32
        )

    return pl.pallas_call(
        body,
        grid=(NBLK,),
        in_specs=[pl.BlockSpec((D, VBLK), lambda i: (0, i))],
        out_specs=pl.BlockSpec((VSUB, 128), lambda i: (i, 0)),
        out_shape=jax.ShapeDtypeStruct((NBLK * VSUB, 128), jnp.int32),
    )(table_t)


def _sc_sums(x_flat, tab32):
    mesh = plsc.VectorSubcoreMesh(
        core_axis_name="c", subcore_axis_name="s", num_cores=NC, num_subcores=NS
    )

    @functools.partial(
        pl.kernel,
        out_type=jax.ShapeDtypeStruct((B, D), jnp.float32),
        mesh=mesh,
        scratch_types=[
            pltpu.VMEM((CHUNK_R,), jnp.int32),
            pltpu.VMEM((CHUNK_R,), jnp.int32),
            pltpu.VMEM((CHUNK_R,), jnp.int32),
            pltpu.VMEM((CHUNK_R,), jnp.int32),
            pltpu.VMEM((CHUNK_R, DW), jnp.int32),
            pltpu.VMEM((CHUNK_R, DW), jnp.int32),
            pltpu.VMEM((PER_W, D), jnp.float32),
            pltpu.SemaphoreType.DMA,
            pltpu.SemaphoreType.DMA,
        ],
        compiler_params=pltpu.CompilerParams(use_tc_tiling_on_sc=False),
    )
    def k(x_hbm, tab_hbm, out_hbm, raw0, raw1, idx0, idx1, rows0, rows1,
          acc_v, sem0, sem1):
        wid = lax.axis_index("s") * NC + lax.axis_index("c")
        base_e = wid * PER_W
        raw_b = (raw0, raw1)
        idx_b = (idx0, idx1)
        rows_b = (rows0, rows1)
        sem_b = (sem0, sem1)

        def start(g, b):
            raw, idx = raw_b[b], idx_b[b]
            pltpu.sync_copy(
                x_hbm.at[pl.ds(base_e * L + g * CHUNK_R, CHUNK_R)], raw
            )

            def tr_body(j, carry):
                r = raw[pl.ds(16 * j, 16)]
                idx[pl.ds(16 * j, 16)] = (
                    (r & ~(VBLK - 1))
                    | ((r & (VSUB - 1)) << 2)
                    | ((r >> 13) & 3)
                )
                return carry

            lax.fori_loop(0, CHUNK_R // 16, tr_body, 0)
            pltpu.async_copy(tab_hbm.at[idx], rows_b[b], sem_b[b])

        def accum(g, b):
            rows = rows_b[b]
            mask = jnp.full((16,), -65536, jnp.int32)

            def l_body(li, accs):
                out = list(accs)
                for e in range(CE):
                    for j in range(2):
                        for q in range(2):
                            v = rows[e * L + 2 * li + j, pl.ds(16 * q, 16)]
                            lo = lax.bitcast_convert_type(v * 65536, jnp.float32)
                            hi = lax.bitcast_convert_type(v & mask, jnp.float32)
                            out[4 * e + 2 * q] = out[4 * e + 2 * q] + lo
                            out[4 * e + 2 * q + 1] = (
                                out[4 * e + 2 * q + 1] + hi
                            )
                return tuple(out)

            zeros = tuple(jnp.zeros((16,), jnp.float32) for _ in range(4 * CE))
            accs = lax.fori_loop(0, L // 2, l_body, zeros)
            for e in range(CE):
                for p in range(4):
                    acc_v[g * CE + e, pl.ds(16 * p, 16)] = accs[4 * e + p]

        start(0, 0)
        start(1, 1)

        def pair_body(gp, carry):
            for b in range(2):
                g = 2 * gp + b
                pltpu.make_async_copy(
                    tab_hbm.at[idx_b[b]], rows_b[b], sem_b[b]
                ).wait()
                accum(g, b)

                @pl.when(g + 2 < NCHUNK)
                def _():
                    start(g + 2, b)

            return carry

        lax.fori_loop(0, NCHUNK // 2, pair_body, 0)
        pltpu.sync_copy(acc_v, out_hbm.at[pl.ds(base_e, PER_W)])

    return k(x_flat, tab32)


def _tc_head(sums, w_perm, b_col):
    def body(w_ref, x_ref, b_ref, o_ref):
        x = x_ref[...] * jnp.float32(1.0 / L)
        o_ref[...] = (
            lax.dot_general(
                w_ref[...], x, (((1,), (1,)), ((), ())),
                preferred_element_type=jnp.float32,
            )
            + b_ref[...]
        )

    mblk = 512
    return pl.pallas_call(
        body,
        grid=(B // mblk,),
        in_specs=[
            pl.BlockSpec((C, D), lambda i: (0, 0)),
            pl.BlockSpec((mblk, D), lambda i: (i, 0)),
            pl.BlockSpec((C, 1), lambda i: (0, 0)),
        ],
        out_specs=pl.BlockSpec((C, mblk), lambda i: (0, i)),
        out_shape=jax.ShapeDtypeStruct((C, B), jnp.float32),
    )(w_perm, sums, b_col)


def kernel(x_data, table, W, b):
    x_flat = x_data.reshape(-1).astype(jnp.int32)
    packed = _tc_pack(table.T)
    tab32 = packed.reshape(NBLK * VBLK, DW)
    sums = _sc_sums(x_flat, tab32)
    pred_t = _tc_head(sums, W[:, _PERM], b.reshape(C, 1))
    return pred_t.T


# speedup vs baseline: 3.8779x; 1.0889x over previous
import functools

import jax
import jax.numpy as jnp
import numpy as np
from jax import lax
from jax.experimental import pallas as pl
from jax.experimental.pallas import tpu as pltpu
from jax.experimental.pallas import tpu_sc as plsc

B = 4096
L = 200
D = 64
DW = D // 2
C = 1000
C_PAD = 1024

V = 1000000
VBLK = 32768
VSUB = VBLK // 4
NBLK = (V + VBLK - 1) // VBLK
V_PAD = NBLK * VBLK

NC, NS = 2, 16
NW = NC * NS
PER_W = B // NW
CE = 4
CHUNK_R = CE * L
NCHUNK = PER_W // CE

_PERM = np.concatenate(
    [np.arange(0, 16), np.arange(32, 48), np.arange(16, 32), np.arange(48, 64)]
)


def _tc_pack(table_t):
    def body(x_ref, o_ref):
        u = lax.bitcast_convert_type(x_ref[...], jnp.uint32)
        lo = (u[:DW, :] + 0x8000) >> 16
        hi = (u[DW:, :] + 0x8000) & jnp.uint32(0xFFFF0000)
        p = lo | hi
        y = jnp.concatenate(
            [p[:, VSUB * c:VSUB * (c + 1)] for c in range(4)], axis=0
        )
        o_ref[...] = lax.bitcast_convert_type(
            lax.transpose(y, (1, 0)), jnp.int32
        )

    return pl.pallas_call(
        body,
        grid=(NBLK,),
        in_specs=[pl.BlockSpec((D, VBLK), lambda i: (0, i))],
        out_specs=pl.BlockSpec((VSUB, 128), lambda i: (i, 0)),
        out_shape=jax.ShapeDtypeStruct((NBLK * VSUB, 128), jnp.int32),
    )(table_t)


def _sc_sums(x_flat, tab32):
    mesh = plsc.VectorSubcoreMesh(
        core_axis_name="c", subcore_axis_name="s", num_cores=NC, num_subcores=NS
    )

    @functools.partial(
        pl.kernel,
        out_type=jax.ShapeDtypeStruct((B, D), jnp.float32),
        mesh=mesh,
        scratch_types=[
            pltpu.VMEM((CHUNK_R,), jnp.int32),
            pltpu.VMEM((CHUNK_R,), jnp.int32),
            pltpu.VMEM((CHUNK_R,), jnp.int32),
            pltpu.VMEM((CHUNK_R,), jnp.int32),
            pltpu.VMEM((CHUNK_R, DW), jnp.int32),
            pltpu.VMEM((CHUNK_R, DW), jnp.int32),
            pltpu.VMEM((PER_W, D), jnp.float32),
            pltpu.SemaphoreType.DMA,
            pltpu.SemaphoreType.DMA,
            pltpu.SemaphoreType.DMA,
            pltpu.SemaphoreType.DMA,
        ],
        compiler_params=pltpu.CompilerParams(use_tc_tiling_on_sc=False),
    )
    def k(x_hbm, tab_hbm, out_hbm, raw0, raw1, idx0, idx1, rows0, rows1,
          acc_v, semg0, semg1, semi0, semi1):
        wid = lax.axis_index("s") * NC + lax.axis_index("c")
        base_e = wid * PER_W
        raw_b = (raw0, raw1)
        idx_b = (idx0, idx1)
        rows_b = (rows0, rows1)
        semg_b = (semg0, semg1)
        semi_b = (semi0, semi1)

        def fetch_idx(g, b):
            pltpu.async_copy(
                x_hbm.at[pl.ds(base_e * L + g * CHUNK_R, CHUNK_R)],
                raw_b[b], semi_b[b],
            )

        def launch(g, b):
            raw, idx = raw_b[b], idx_b[b]
            pltpu.make_async_copy(
                x_hbm.at[pl.ds(base_e * L + g * CHUNK_R, CHUNK_R)],
                raw, semi_b[b],
            ).wait()

            def tr_body(j, carry):
                r = raw[pl.ds(16 * j, 16)]
                idx[pl.ds(16 * j, 16)] = (
                    (r & ~(VBLK - 1))
                    | ((r & (VSUB - 1)) << 2)
                    | ((r >> 13) & 3)
                )
                return carry

            lax.fori_loop(0, CHUNK_R // 16, tr_body, 0)
            pltpu.async_copy(tab_hbm.at[idx], rows_b[b], semg_b[b])

        def accum(g, b):
            rows = rows_b[b]

            def l_body(li, accs):
                out = list(accs)
                for e in range(CE):
                    for j in range(2):
                        for q in range(2):
                            v = rows[e * L + 2 * li + j, pl.ds(16 * q, 16)]
                            lo = lax.bitcast_convert_type(v * 65536, jnp.float32)
                            hi = lax.bitcast_convert_type(v, jnp.float32)
                            out[4 * e + 2 * q] = out[4 * e + 2 * q] + lo
                            out[4 * e + 2 * q + 1] = (
                                out[4 * e + 2 * q + 1] + hi
                            )
                return tuple(out)

            zeros = tuple(jnp.zeros((16,), jnp.float32) for _ in range(4 * CE))
            accs = lax.fori_loop(0, L // 2, l_body, zeros)
            for e in range(CE):
                for p in range(4):
                    acc_v[g * CE + e, pl.ds(16 * p, 16)] = accs[4 * e + p]

        fetch_idx(0, 0)
        fetch_idx(1, 1)
        launch(0, 0)
        fetch_idx(2, 0)
        launch(1, 1)
        fetch_idx(3, 1)

        def pair_body(gp, carry):
            for b in range(2):
                g = 2 * gp + b
                pltpu.make_async_copy(
                    tab_hbm.at[idx_b[b]], rows_b[b], semg_b[b]
                ).wait()
                accum(g, b)

                @pl.when(g + 2 < NCHUNK)
                def _():
                    launch(g + 2, b)

                @pl.when(g + 4 < NCHUNK)
                def _():
                    fetch_idx(g + 4, b)

            return carry

        lax.fori_loop(0, NCHUNK // 2, pair_body, 0)
        pltpu.sync_copy(acc_v, out_hbm.at[pl.ds(base_e, PER_W)])

    return k(x_flat, tab32)


def _tc_head(sums, w_perm, b_col):
    def body(w_ref, x_ref, b_ref, o_ref):
        x = x_ref[...] * jnp.float32(1.0 / L)
        o_ref[...] = (
            lax.dot_general(
                w_ref[...], x, (((1,), (1,)), ((), ())),
                preferred_element_type=jnp.float32,
            )
            + b_ref[...]
        )

    mblk = 512
    return pl.pallas_call(
        body,
        grid=(B // mblk,),
        in_specs=[
            pl.BlockSpec((C, D), lambda i: (0, 0)),
            pl.BlockSpec((mblk, D), lambda i: (i, 0)),
            pl.BlockSpec((C, 1), lambda i: (0, 0)),
        ],
        out_specs=pl.BlockSpec((C, mblk), lambda i: (0, i)),
        out_shape=jax.ShapeDtypeStruct((C, B), jnp.float32),
    )(w_perm, sums, b_col)


def kernel(x_data, table, W, b):
    x_flat = x_data.reshape(-1).astype(jnp.int32)
    packed = _tc_pack(table.T)
    tab32 = packed.reshape(NBLK * VBLK, DW)
    sums = _sc_sums(x_flat, tab32)
    pred_t = _tc_head(sums, W[:, _PERM], b.reshape(C, 1))
    return pred_t.T
